# baseline (speedup 1.0000x reference)
import dataclasses
import functools

import jax
import jax.numpy as jnp
import numpy as np
from jax import lax
from jax.experimental import pallas as pl
from jax.experimental.pallas import tpu as pltpu
from jax.experimental.pallas import tpu_sc as plsc

N_ROWS = 16384
N_CLASSES = 1000
NBINS = 10
ROW_BLOCK = 1024

SC_ROWS = 128
SC_BLK = 16
_SC_SUBS = 32
_NCHUNK = N_CLASSES // 16
_TAIL_START = N_CLASSES - 16

_BOUNDS = np.array(
    [0x00000000, 0x3DCCCCCD, 0x3E4CCCCD, 0x3E99999A, 0x3ECCCCCD, 0x3F000000,
     0x3F19999A, 0x3F333333, 0x3F4CCCCD, 0x3F666667, 0x3F800000],
    dtype=np.uint32,
).view(np.float32)


def _ece_tc_kernel(x_ref, t_ref, out_ref, hist_ref):
    i = pl.program_id(0)
    nsteps = pl.num_programs(0)

    x = x_ref[...]
    t = t_ref[...]
    R, C = x.shape

    col = lax.broadcasted_iota(jnp.int32, (R, C), 1)
    m = jnp.max(x, axis=1, keepdims=True)
    e = jnp.exp(x - m)
    s = jnp.sum(e, axis=1, keepdims=True)
    te = jnp.sum(jnp.where(col == t, e, 0.0), axis=1, keepdims=True)
    p = te / s
    am = jnp.min(jnp.where(x == m, col, C), axis=1, keepdims=True)
    correct = (am == t).astype(jnp.float32)

    lane = lax.broadcasted_iota(jnp.int32, (1, 128), 1)
    lo = jnp.full((1, 128), 2.0, jnp.float32)
    hi = jnp.full((1, 128), 3.0, jnp.float32)
    for k in range(NBINS):
        lo = jnp.where(lane == k, float(_BOUNDS[k]), lo)
        hi = jnp.where(lane == k, float(_BOUNDS[k + 1]), hi)
    onehot = ((p > lo) & (p <= hi)).astype(jnp.float32)
    cnt = jnp.sum(onehot, axis=0, keepdims=True)
    sp = jnp.sum(onehot * p, axis=0, keepdims=True)
    sc = jnp.sum(onehot * correct, axis=0, keepdims=True)

    @pl.when(i == 0)
    def _():
        hist_ref[...] = jnp.zeros_like(hist_ref)

    hist_ref[0:1, :] += cnt
    hist_ref[1:2, :] += sp
    hist_ref[2:3, :] += sc

    @pl.when(i == nsteps - 1)
    def _():
        out_ref[...] = hist_ref[...]


def _tc_main(outputs, t2d):
    rows = N_ROWS - SC_ROWS
    off = SC_ROWS // ROW_BLOCK
    return pl.pallas_call(
        _ece_tc_kernel,
        grid=(rows // ROW_BLOCK,),
        in_specs=[
            pl.BlockSpec((ROW_BLOCK, N_CLASSES), lambda i: (i + off, 0)),
            pl.BlockSpec((ROW_BLOCK, 1), lambda i: (i, 0)),
        ],
        out_specs=pl.BlockSpec((8, 128), lambda i: (0, 0)),
        out_shape=jax.ShapeDtypeStruct((8, 128), jnp.float32),
        scratch_shapes=[pltpu.VMEM((8, 128), jnp.float32)],
    )(outputs, t2d)


def _sc_slice(x, t_sc):
    cp = pltpu.CompilerParams()
    if "needs_layout_passes" in pltpu.CompilerParams.__dataclass_fields__:
        cp = dataclasses.replace(cp, needs_layout_passes=False)

    @pl.kernel(
        out_type=jax.ShapeDtypeStruct((_SC_SUBS, 3, 16), jnp.float32),
        mesh=plsc.VectorSubcoreMesh(core_axis_name="c", subcore_axis_name="s"),
        scratch_types=[
            pltpu.VMEM((3, 16), jnp.float32),
            pltpu.SemaphoreType.DMA,
        ],
        compiler_params=cp,
    )
    def k(x_hbm, t_hbm, o_hbm, acc, sem):
        sub = lax.axis_index("c") * 16 + lax.axis_index("s")
        lanei = lax.broadcasted_iota(jnp.int32, (16,), 0)
        lanef = lanei.astype(jnp.float32)
        lov = jnp.full((16,), 2.0, jnp.float32)
        hiv = jnp.full((16,), 3.0, jnp.float32)
        for kk in range(NBINS):
            lov = jnp.where(lanei == kk, float(_BOUNDS[kk]), lov)
            hiv = jnp.where(lanei == kk, float(_BOUNDS[kk + 1]), hiv)
        for kk in range(3):
            acc[kk, :] = jnp.zeros((16,), jnp.float32)

        def body(x_vmem, t_vmem):
            tall = t_vmem[0, pl.ds(0, 16)]
            for r in range(SC_BLK):
                t_r = tall[r]
                def mx(i, mcarry):
                    v = x_vmem[r, pl.ds(i * 16, 16)]
                    return jnp.maximum(mcarry, v)
                m_acc = lax.fori_loop(0, _NCHUNK, mx, jnp.full((16,), -jnp.inf, jnp.float32), unroll=4)
                tv = x_vmem[r, pl.ds(_TAIL_START, 16)]
                m_acc = jnp.maximum(m_acc, jnp.where(lanei >= 8, tv, -jnp.inf))
                m = jnp.max(m_acc)

                def ex(i, carry):
                    s_acc, te_acc, am_acc = carry
                    v = x_vmem[r, pl.ds(i * 16, 16)]
                    e = jnp.exp(v - m)
                    idx = lanei + i * 16
                    am_acc = jnp.minimum(am_acc, jnp.where(v == m, idx, N_CLASSES))
                    te_acc = te_acc + jnp.where(idx == t_r, e, 0.0)
                    return s_acc + e, te_acc, am_acc
                s_acc, te_acc, am_acc = lax.fori_loop(
                    0, _NCHUNK, ex,
                    (jnp.zeros((16,), jnp.float32), jnp.zeros((16,), jnp.float32),
                     jnp.full((16,), N_CLASSES, jnp.int32)), unroll=4)
                valid = lanei >= 8
                ev = jnp.exp(tv - m)
                idxt = lanei + _TAIL_START
                s_acc = s_acc + jnp.where(valid, ev, 0.0)
                am_acc = jnp.minimum(
                    am_acc, jnp.where(valid & (tv == m), idxt, N_CLASSES))
                te_acc = te_acc + jnp.where(valid & (idxt == t_r), ev, 0.0)

                s = jnp.sum(s_acc)
                te = jnp.sum(te_acc)
                am = jnp.min(am_acc)
                p16 = jnp.broadcast_to(te, (16,)) / jnp.broadcast_to(s, (16,))
                corr = (am == t_r).astype(jnp.float32)

                onehot = ((p16 > lov) & (p16 <= hiv)).astype(jnp.float32)
                acc[0, :] += onehot
                acc[1, :] += onehot * p16
                acc[2, :] += onehot * corr

        pltpu.emit_pipeline(
            body,
            grid=(SC_ROWS // SC_BLK,),
            in_specs=[
                pl.BlockSpec((SC_BLK, N_CLASSES), lambda i: (i, 0)),
                pl.BlockSpec((1, 128), lambda i: (i, 0)),
            ],
            out_specs=[],
            core_axis_name=("c", "s"),
            dimension_semantics=(pltpu.PARALLEL,),
        )(x_hbm, t_hbm)

        pltpu.sync_copy(acc, o_hbm.at[sub])

    return k(x, t_sc)


def _combine_kernel(tc_ref, sc_ref, out_ref):
    tc = tc_ref[...]
    scp = jnp.sum(sc_ref[...], axis=0, keepdims=True)
    cnt = tc[0:1, 0:16] + scp[:, 0:16]
    sp = tc[1:2, 0:16] + scp[:, 16:32]
    sc = tc[2:3, 0:16] + scp[:, 32:48]
    safe = jnp.maximum(cnt, 1.0)
    term = jnp.where(cnt > 0, cnt * jnp.abs(sp / safe - sc / safe), 0.0)
    total = jnp.sum(cnt, keepdims=True)
    ece = jnp.where(total > 0, jnp.sum(term, keepdims=True) / total, 0.0)
    out_ref[...] = ece


def _combine(tc_hist, sc_parts):
    return pl.pallas_call(
        _combine_kernel,
        grid=(1,),
        in_specs=[
            pl.BlockSpec((8, 128), lambda i: (0, 0)),
            pl.BlockSpec((_SC_SUBS, 48), lambda i: (0, 0)),
        ],
        out_specs=pl.BlockSpec((1, 1), lambda i: (0, 0)),
        out_shape=jax.ShapeDtypeStruct((1, 1), jnp.float32),
    )(tc_hist, sc_parts)


@jax.jit
def _ece(outputs, targets):
    t32 = targets.astype(jnp.int32)
    t_sc = jnp.pad(t32[:SC_ROWS].reshape(SC_ROWS // SC_BLK, SC_BLK),
                   ((0, 0), (0, 128 - SC_BLK)))
    t_tc = t32[SC_ROWS:].reshape(N_ROWS - SC_ROWS, 1)
    sc_parts = _sc_slice(outputs, t_sc)
    tc_hist = _tc_main(outputs, t_tc)
    ece = _combine(tc_hist, sc_parts.reshape(_SC_SUBS, 48))
    return ece.reshape(())


def kernel(outputs, targets):
    return _ece(outputs, targets)


# speedup vs baseline: 1.2329x; 1.2329x over previous
import functools

import jax
import jax.numpy as jnp
import numpy as np
from jax import lax
from jax.experimental import pallas as pl
from jax.experimental.pallas import tpu as pltpu

N_ROWS = 16384
N_CLASSES = 1000
NBINS = 10
ROW_BLOCK = 2048

_BOUNDS = np.array(
    [0x00000000, 0x3DCCCCCD, 0x3E4CCCCD, 0x3E99999A, 0x3ECCCCCD, 0x3F000000,
     0x3F19999A, 0x3F333333, 0x3F4CCCCD, 0x3F666667, 0x3F800000],
    dtype=np.uint32,
).view(np.float32)


def _ece_tc_kernel(x_ref, t_ref, out_ref, hist_ref):
    i = pl.program_id(0)
    nsteps = pl.num_programs(0)

    x = x_ref[...]
    t = t_ref[...]
    R, C = x.shape

    col = lax.broadcasted_iota(jnp.int32, (R, C), 1)
    m = jnp.max(x, axis=1, keepdims=True)
    e = jnp.exp(x - m)
    s = jnp.sum(e, axis=1, keepdims=True)
    te = jnp.sum(jnp.where(col == t, e, 0.0), axis=1, keepdims=True)
    p = te / s
    am = jnp.min(jnp.where(x == m, col, C), axis=1, keepdims=True)
    correct = (am == t).astype(jnp.float32)

    lane = lax.broadcasted_iota(jnp.int32, (1, 128), 1)
    lo = jnp.full((1, 128), 2.0, jnp.float32)
    hi = jnp.full((1, 128), 3.0, jnp.float32)
    for k in range(NBINS):
        lo = jnp.where(lane == k, float(_BOUNDS[k]), lo)
        hi = jnp.where(lane == k, float(_BOUNDS[k + 1]), hi)
    onehot = ((p > lo) & (p <= hi)).astype(jnp.float32)
    cnt = jnp.sum(onehot, axis=0, keepdims=True)
    sp = jnp.sum(onehot * p, axis=0, keepdims=True)
    sc = jnp.sum(onehot * correct, axis=0, keepdims=True)

    @pl.when(i == 0)
    def _():
        hist_ref[...] = jnp.zeros_like(hist_ref)

    hist_ref[0:1, :] += cnt
    hist_ref[1:2, :] += sp
    hist_ref[2:3, :] += sc

    @pl.when(i == nsteps - 1)
    def _():
        cntv = hist_ref[0:1, :]
        spv = hist_ref[1:2, :]
        scv = hist_ref[2:3, :]
        safe = jnp.maximum(cntv, 1.0)
        term = jnp.where(cntv > 0, cntv * jnp.abs(spv / safe - scv / safe), 0.0)
        total = jnp.sum(cntv, keepdims=True)
        ece = jnp.where(total > 0, jnp.sum(term, keepdims=True) / total, 0.0)
        out_ref[...] = ece


@jax.jit
def _ece(outputs, targets):
    t2d = targets.astype(jnp.int32).reshape(N_ROWS, 1)
    out = pl.pallas_call(
        _ece_tc_kernel,
        grid=(N_ROWS // ROW_BLOCK,),
        in_specs=[
            pl.BlockSpec((ROW_BLOCK, N_CLASSES), lambda i: (i, 0)),
            pl.BlockSpec((ROW_BLOCK, 1), lambda i: (i, 0)),
        ],
        out_specs=pl.BlockSpec((1, 1), lambda i: (0, 0)),
        out_shape=jax.ShapeDtypeStruct((1, 1), jnp.float32),
        scratch_shapes=[pltpu.VMEM((8, 128), jnp.float32)],
    )(outputs, t2d)
    return out.reshape(())


def kernel(outputs, targets):
    return _ece(outputs, targets)
